# baseline (speedup 1.0000x reference)
import jax
import jax.numpy as jnp
from jax.experimental import pallas as pl
from jax.experimental.pallas import tpu as pltpu

N = 1024
IN_FEAT = 128
N_HIDDEN = 64
N_HEADS = 8
FCAT = N_HIDDEN * N_HEADS
OUT_FEAT = 64
NEG_SLOPE = 0.2


def _mask01_rows(adj_rows, row0):
    nrows = adj_rows.shape[0]
    ii = jax.lax.broadcasted_iota(jnp.int32, (nrows, N), 0) + row0
    jj = jax.lax.broadcasted_iota(jnp.int32, (nrows, N), 1)
    valid = ((adj_rows != 0) & (ii != jj)) | (ii == jj)
    return jnp.where(valid, 1.0, 0.0)


def _attend(A, C, E, mask01, h_ext):
    q = jnp.maximum(A, C * E) * mask01
    return jax.lax.dot_general(h_ext, q, (((0,), (0,)), ((), ())),
                               preferred_element_type=jnp.float32)


def _blockdiag(col):
    r = jax.lax.broadcasted_iota(jnp.int32, (FCAT, N_HEADS), 0)
    c = jax.lax.broadcasted_iota(jnp.int32, (FCAT, N_HEADS), 1)
    return jnp.where((r // N_HIDDEN) == c, col, 0.0)


def _fused_kernel(x_ref, adj_ref, Wh_ref, asrc_ref, adst_ref, b_ref,
                  Wout_ref, as2_ref, ad2_ref, b2_ref, out_ref,
                  hcatT_scr, adj_vmem, sems):
    half = N // 2
    cp0 = pltpu.make_async_copy(adj_ref.at[pl.ds(0, half), :],
                                adj_vmem.at[pl.ds(0, half), :], sems.at[0])
    cp1 = pltpu.make_async_copy(adj_ref.at[pl.ds(half, half), :],
                                adj_vmem.at[pl.ds(half, half), :], sems.at[1])
    cp0.start()
    cp1.start()

    Wcat = jnp.concatenate([Wh_ref[k] for k in range(N_HEADS)], axis=1)
    hproj = jnp.dot(x_ref[...], Wcat,
                    preferred_element_type=jnp.float32)
    s_all = jnp.dot(hproj, _blockdiag(asrc_ref[...]),
                    preferred_element_type=jnp.float32)
    d_allT = jax.lax.dot_general(
        _blockdiag(adst_ref[...]), hproj, (((0,), (1,)), ((), ())),
        preferred_element_type=jnp.float32)
    bT = jnp.transpose(b_ref[...])

    cp0.wait()
    m0 = _mask01_rows(adj_vmem[0:half, :], 0)
    cp1.wait()
    m1 = _mask01_rows(adj_vmem[half:N, :], half)
    mask01 = jnp.concatenate([m0, m1], axis=0)
    ones_col = jnp.ones((N, 1), dtype=jnp.float32)

    smax_all = jnp.max(s_all, axis=0, keepdims=True)
    A_all = jnp.exp(s_all - smax_all)
    C_all = jnp.exp(NEG_SLOPE * (s_all - smax_all))

    for k in range(N_HEADS):
        E = jnp.exp(-(1.0 - NEG_SLOPE) *
                    (d_allT[k:k + 1, :] + smax_all[:, k:k + 1]))
        h_ext = jnp.concatenate(
            [hproj[:, k * N_HIDDEN:(k + 1) * N_HIDDEN], ones_col], axis=1)
        accT = _attend(A_all[:, k:k + 1], C_all[:, k:k + 1], E,
                       mask01, h_ext)
        recip = 1.0 / (accT[N_HIDDEN:N_HIDDEN + 1, :] + 1e-16)
        outT = accT[:N_HIDDEN, :] * recip + bT[:, k:k + 1]
        hcatT_scr[k * N_HIDDEN:(k + 1) * N_HIDDEN, :] = outT

    hcT = hcatT_scr[...]
    h2 = jax.lax.dot_general(hcT, Wout_ref[...], (((0,), (0,)), ((), ())),
                             preferred_element_type=jnp.float32)
    s2 = jax.lax.dot_general(h2, as2_ref[...], (((1,), (1,)), ((), ())),
                             preferred_element_type=jnp.float32)
    d2 = jax.lax.dot_general(ad2_ref[...], h2, (((1,), (1,)), ((), ())),
                             preferred_element_type=jnp.float32)
    smax2 = jnp.max(s2, axis=0, keepdims=True)
    A2 = jnp.exp(s2 - smax2)
    C2 = jnp.exp(NEG_SLOPE * (s2 - smax2))
    E2 = jnp.exp(-(1.0 - NEG_SLOPE) * (d2 + smax2))
    h2_ext = jnp.concatenate([h2, ones_col], axis=1)
    accT2 = _attend(A2, C2, E2, mask01, h2_ext)
    recip2 = 1.0 / (accT2[OUT_FEAT:OUT_FEAT + 1, :] + 1e-16)
    oT = accT2[:OUT_FEAT, :] * recip2 + jnp.transpose(b2_ref[...])
    oT = jnp.where(oT > 0, oT, jnp.exp(oT) - 1.0)
    mm = jnp.max(oT, axis=0, keepdims=True)
    z = oT - mm
    lse = jnp.log(jnp.sum(jnp.exp(z), axis=0, keepdims=True))
    out_ref[...] = jnp.transpose(z - lse)


def _full_spec(*shape):
    n = len(shape)
    return pl.BlockSpec(shape, lambda: (0,) * n)


def kernel(x, adj, W_heads, a_src_heads, a_dst_heads, b_heads,
           W_out, a_src_out, a_dst_out, b_out):
    adj = adj.astype(jnp.int32)

    out = pl.pallas_call(
        _fused_kernel,
        in_specs=[
            _full_spec(N, IN_FEAT),
            pl.BlockSpec(memory_space=pltpu.MemorySpace.HBM),
            _full_spec(N_HEADS, IN_FEAT, N_HIDDEN),
            _full_spec(FCAT, 1),
            _full_spec(FCAT, 1),
            _full_spec(N_HEADS, N_HIDDEN),
            _full_spec(FCAT, OUT_FEAT),
            _full_spec(1, OUT_FEAT),
            _full_spec(1, OUT_FEAT),
            _full_spec(1, OUT_FEAT),
        ],
        out_specs=_full_spec(N, OUT_FEAT),
        out_shape=jax.ShapeDtypeStruct((N, OUT_FEAT), jnp.float32),
        scratch_shapes=[pltpu.VMEM((FCAT, N), jnp.float32),
                        pltpu.VMEM((N, N), jnp.int32),
                        pltpu.SemaphoreType.DMA((2,))],
    )(x, adj, W_heads,
      a_src_heads.reshape(FCAT, 1), a_dst_heads.reshape(FCAT, 1), b_heads,
      W_out,
      a_src_out.reshape(1, OUT_FEAT), a_dst_out.reshape(1, OUT_FEAT),
      b_out.reshape(1, OUT_FEAT))
    return out


# speedup vs baseline: 1.0574x; 1.0574x over previous
import jax
import jax.numpy as jnp
from jax.experimental import pallas as pl
from jax.experimental.pallas import tpu as pltpu

N = 1024
IN_FEAT = 128
N_HIDDEN = 64
N_HEADS = 8
FCAT = N_HIDDEN * N_HEADS
OUT_FEAT = 64
NEG_SLOPE = 0.2


def _mask01(adj):
    ii = jax.lax.broadcasted_iota(jnp.int32, (N, N), 0)
    jj = jax.lax.broadcasted_iota(jnp.int32, (N, N), 1)
    return ((adj != 0) & (ii != jj)) | (ii == jj)


def _attend(A, C, E, mask01, h_ext):
    q = jnp.where(mask01, jnp.maximum(A, C * E), 0.0)
    return jax.lax.dot_general(h_ext, q, (((0,), (0,)), ((), ())),
                               preferred_element_type=jnp.float32)


def _blockdiag(col):
    r = jax.lax.broadcasted_iota(jnp.int32, (FCAT, N_HEADS), 0)
    c = jax.lax.broadcasted_iota(jnp.int32, (FCAT, N_HEADS), 1)
    return jnp.where((r // N_HIDDEN) == c, col, 0.0)


def _fused_kernel(x_ref, adj_ref, Wh_ref, asrc_ref, adst_ref, b_ref,
                  Wout_ref, as2_ref, ad2_ref, b2_ref, out_ref, hcatT_scr):
    Wcat = jnp.concatenate([Wh_ref[k] for k in range(N_HEADS)], axis=1)
    hproj = jnp.dot(x_ref[...], Wcat,
                    preferred_element_type=jnp.float32)
    s_all = jnp.dot(hproj, _blockdiag(asrc_ref[...]),
                    preferred_element_type=jnp.float32)
    d_allT = jax.lax.dot_general(
        _blockdiag(adst_ref[...]), hproj, (((0,), (1,)), ((), ())),
        preferred_element_type=jnp.float32)
    bT = jnp.transpose(b_ref[...])
    mask01 = _mask01(adj_ref[...])
    ones_col = jnp.ones((N, 1), dtype=jnp.float32)

    smax_all = jnp.max(s_all, axis=0, keepdims=True)
    A_all = jnp.exp(s_all - smax_all)
    C_all = jnp.exp(NEG_SLOPE * (s_all - smax_all))

    for k in range(N_HEADS):
        E = jnp.exp(-(1.0 - NEG_SLOPE) *
                    (d_allT[k:k + 1, :] + smax_all[:, k:k + 1]))
        h_ext = jnp.concatenate(
            [hproj[:, k * N_HIDDEN:(k + 1) * N_HIDDEN], ones_col], axis=1)
        accT = _attend(A_all[:, k:k + 1], C_all[:, k:k + 1], E,
                       mask01, h_ext)
        recip = 1.0 / (accT[N_HIDDEN:N_HIDDEN + 1, :] + 1e-16)
        outT = accT[:N_HIDDEN, :] * recip + bT[:, k:k + 1]
        hcatT_scr[k * N_HIDDEN:(k + 1) * N_HIDDEN, :] = outT

    hcT = hcatT_scr[...]
    h2 = jax.lax.dot_general(hcT, Wout_ref[...], (((0,), (0,)), ((), ())),
                             preferred_element_type=jnp.float32)
    s2 = jax.lax.dot_general(h2, as2_ref[...], (((1,), (1,)), ((), ())),
                             preferred_element_type=jnp.float32)
    d2 = jax.lax.dot_general(ad2_ref[...], h2, (((1,), (1,)), ((), ())),
                             preferred_element_type=jnp.float32)
    smax2 = jnp.max(s2, axis=0, keepdims=True)
    A2 = jnp.exp(s2 - smax2)
    C2 = jnp.exp(NEG_SLOPE * (s2 - smax2))
    E2 = jnp.exp(-(1.0 - NEG_SLOPE) * (d2 + smax2))
    h2_ext = jnp.concatenate([h2, ones_col], axis=1)
    accT2 = _attend(A2, C2, E2, mask01, h2_ext)
    recip2 = 1.0 / (accT2[OUT_FEAT:OUT_FEAT + 1, :] + 1e-16)
    oT = accT2[:OUT_FEAT, :] * recip2 + jnp.transpose(b2_ref[...])
    oT = jnp.where(oT > 0, oT, jnp.exp(oT) - 1.0)
    mm = jnp.max(oT, axis=0, keepdims=True)
    z = oT - mm
    lse = jnp.log(jnp.sum(jnp.exp(z), axis=0, keepdims=True))
    out_ref[...] = jnp.transpose(z - lse)


def _full_spec(*shape):
    n = len(shape)
    return pl.BlockSpec(shape, lambda: (0,) * n)


def kernel(x, adj, W_heads, a_src_heads, a_dst_heads, b_heads,
           W_out, a_src_out, a_dst_out, b_out):
    adj = adj.astype(jnp.int32)

    out = pl.pallas_call(
        _fused_kernel,
        in_specs=[
            _full_spec(N, IN_FEAT),
            _full_spec(N, N),
            _full_spec(N_HEADS, IN_FEAT, N_HIDDEN),
            _full_spec(FCAT, 1),
            _full_spec(FCAT, 1),
            _full_spec(N_HEADS, N_HIDDEN),
            _full_spec(FCAT, OUT_FEAT),
            _full_spec(1, OUT_FEAT),
            _full_spec(1, OUT_FEAT),
            _full_spec(1, OUT_FEAT),
        ],
        out_specs=_full_spec(N, OUT_FEAT),
        out_shape=jax.ShapeDtypeStruct((N, OUT_FEAT), jnp.float32),
        scratch_shapes=[pltpu.VMEM((FCAT, N), jnp.float32)],
    )(x, adj, W_heads,
      a_src_heads.reshape(FCAT, 1), a_dst_heads.reshape(FCAT, 1), b_heads,
      W_out,
      a_src_out.reshape(1, OUT_FEAT), a_dst_out.reshape(1, OUT_FEAT),
      b_out.reshape(1, OUT_FEAT))
    return out
